# Initial kernel scaffold; baseline (speedup 1.0000x reference)
import jax
import jax.numpy as jnp
from jax.experimental import pallas as pl


def kernel(input, W1, W2, W3, W4, W5, g5, b5, m5, v5, W6, g6, b6, m6, v6, W7, b7):
    raise NotImplementedError("write your pallas kernel here")


# speedup vs baseline: 8.7453x; 8.7453x over previous
import jax
import jax.numpy as jnp
from jax.experimental import pallas as pl

_K = 10
_N = 220
_HI = jax.lax.Precision.HIGHEST


def _lrelu(v):
    return jnp.where(v >= 0, v, 0.2 * v)


def _edge_layer(x, w_t):
    g = jax.lax.dot_general(x, x, (((1,), (1,)), ((), ())))
    xx = jnp.sum(x * x, axis=1)
    d = 2.0 * g - xx[:, None] - xx[None, :]
    col = jax.lax.broadcasted_iota(jnp.int32, (_N, _N), 1)
    acc = None
    for _ in range(_K):
        m = jnp.max(d, axis=1, keepdims=True)
        a = jnp.min(jnp.where(d == m, col, _N), axis=1, keepdims=True)
        sel = col == a
        nbr = jnp.dot(sel.astype(jnp.float32), x, precision=_HI)
        feat = jnp.concatenate([nbr - x, x], axis=1)
        c = _lrelu(jnp.dot(feat, w_t))
        acc = c if acc is None else jnp.maximum(acc, c)
        d = jnp.where(sel, -jnp.inf, d)
    return acc


def _backbone_step(x_ref, w1t, w2t, w3t, w4t, w5t, s5, t5, z_ref):
    x = x_ref[0]
    x1 = _edge_layer(x, w1t[...])
    x2 = _edge_layer(x1, w2t[...])
    x3 = _edge_layer(x2, w3t[...])
    x4 = _edge_layer(x3, w4t[...])
    xc = jnp.concatenate([x1, x2, x3, x4], axis=1)
    h = jnp.dot(xc, w5t[...])
    h = _lrelu(h * s5[...] + t5[...])
    p1 = jnp.max(h, axis=0)
    p2 = jnp.sum(h, axis=0) * (1.0 / _N)
    z_ref[...] = jnp.concatenate([p1, p2])[None, None, :]


def _head_step(z_ref, w6t, s6, t6, w7t, b7, o_ref):
    u = jnp.dot(z_ref[...], w6t[...])
    u = _lrelu(u * s6[...] + t6[...])
    o_ref[...] = jnp.dot(u, w7t[...]) + b7[...]


def _full(w):
    return pl.BlockSpec(w.shape, lambda i: (0,) * w.ndim)


def kernel(input, W1, W2, W3, W4, W5, g5, b5, m5, v5, W6, g6, b6, m6, v6,
           W7, b7):
    x = input.reshape(-1, _N, 3)
    B = x.shape[0]

    s5 = (g5 / jnp.sqrt(v5 + 1e-5))[None, :]
    t5 = (b5 - m5 * (g5 / jnp.sqrt(v5 + 1e-5)))[None, :]
    s6 = (g6 / jnp.sqrt(v6 + 1e-5))[None, :]
    t6 = (b6 - m6 * (g6 / jnp.sqrt(v6 + 1e-5)))[None, :]
    weights = (W1.T, W2.T, W3.T, W4.T, W5.T, s5, t5)

    z = pl.pallas_call(
        _backbone_step,
        grid=(B,),
        in_specs=[pl.BlockSpec((1, _N, 3), lambda i: (i, 0, 0))]
        + [_full(w) for w in weights],
        out_specs=pl.BlockSpec((1, 1, 1024), lambda i: (i, 0, 0)),
        out_shape=jax.ShapeDtypeStruct((B, 1, 1024), jnp.float32),
    )(x, *weights)
    z = z.reshape(B, 1024)

    head_w = (W6.T, s6, t6, W7.T, b7[None, :])
    out = pl.pallas_call(
        _head_step,
        in_specs=[pl.BlockSpec(z.shape, None)]
        + [pl.BlockSpec(w.shape, None) for w in head_w],
        out_specs=pl.BlockSpec((B, 380), None),
        out_shape=jax.ShapeDtypeStruct((B, 380), jnp.float32),
    )(z, *head_w)
    return out
